# baseline (speedup 1.0000x reference)
import jax
import jax.numpy as jnp
from jax.experimental import pallas as pl
from jax.experimental.pallas import tpu as pltpu

_NTOKEN = 50000
_NINP = 256
_NHID = 256
_NHEADS = 8
_NLAYERS = 6
_B = 100
_NPG = 100
_EPG = 1600
_HD = _NHID // _NHEADS
_G = 5


def _gnn_graph_kernel(src_ref, dst_ref, x_ref, w_ref, as_ref, ad_ref, b_ref,
                      out_ref, acc_ref):

    src = src_ref[...]
    dst = dst_ref[...]

    node_iota = jax.lax.broadcasted_iota(jnp.int32, (_G, _NPG, _EPG), 1)
    st = (node_iota == src).astype(jnp.bfloat16)
    dt = (node_iota == dst).astype(jnp.bfloat16)
    cnt = jax.lax.dot_general(dt, st, (((2,), (2,)), ((0,), (0,))),
                              preferred_element_type=jnp.float32)
    eye_r = jax.lax.broadcasted_iota(jnp.int32, (_G, _NPG, _NPG), 1)
    eye_c = jax.lax.broadcasted_iota(jnp.int32, (_G, _NPG, _NPG), 2)
    cnt = cnt + (eye_r == eye_c).astype(jnp.float32)

    ones_col = jnp.ones((_NPG, 1), dtype=jnp.bfloat16)
    ones_row = jnp.ones((1, _NPG), dtype=jnp.bfloat16)
    xs = [x_ref[gi] for gi in range(_G)]
    for l in range(_NLAYERS):
        w = w_ref[l]
        a_s = as_ref[l]
        a_d = ad_ref[l]
        for gi in range(_G):
            x = xs[gi]
            if l > 0:
                x = jnp.maximum(x, 0.0)
            h = jnp.dot(x.astype(jnp.bfloat16), w,
                        preferred_element_type=jnp.float32)
            hb = h.astype(jnp.bfloat16)
            als_t = jax.lax.dot_general(a_s, hb, (((0,), (1,)), ((), ())),
                                        preferred_element_type=jnp.float32)
            ald = jnp.dot(hb, a_d, preferred_element_type=jnp.float32)
            aldb = ald.astype(jnp.bfloat16)
            for hd_i in range(_NHEADS):
                ald_bc = jnp.dot(aldb[:, hd_i:hd_i + 1], ones_row,
                                 preferred_element_type=jnp.float32)
                e = ald_bc + als_t[hd_i:hd_i + 1, :]
                e = jnp.where(e > 0, e, 0.2 * e)
                ee = (cnt[gi] * jnp.exp(e)).astype(jnp.bfloat16)
                h_aug = jnp.concatenate(
                    [hb[:, hd_i * _HD:(hd_i + 1) * _HD], ones_col], axis=1)
                agg = jnp.dot(ee, h_aug, preferred_element_type=jnp.float32)
                acc_ref[gi, :, hd_i * _HD:(hd_i + 1) * _HD] = (
                    agg[:, :_HD] / agg[:, _HD:_HD + 1])
            xs[gi] = acc_ref[gi] + b_ref[l]

    for gi in range(_G):
        out_ref[gi] = jnp.sum(xs[gi], axis=0, keepdims=True) * (1.0 / _NPG)


def kernel(nodes, edges, emb,
           W0, as0, ad0, b0, W1, as1, ad1, b1, W2, as2, ad2, b2,
           W3, as3, ad3, b3, W4, as4, ad4, b4, W5, as5, ad5, b5):
    x = emb[nodes.reshape(-1)].reshape(_B, _NPG, _NINP)
    src = edges[:, 0::2].reshape(_B, 1, _EPG)
    dst = edges[:, 1::2].reshape(_B, 1, _EPG)

    w_all = jnp.stack([W0, W1, W2, W3, W4, W5]).astype(jnp.bfloat16)
    eye_h = jnp.eye(_NHEADS, dtype=jnp.float32)
    as_all = jnp.stack([
        jnp.einsum('hj,hk->hjk', a, eye_h).reshape(_NHID, _NHEADS)
        for a in (as0, as1, as2, as3, as4, as5)]).astype(jnp.bfloat16)
    ad_all = jnp.stack([
        jnp.einsum('hj,hk->hjk', a, eye_h).reshape(_NHID, _NHEADS)
        for a in (ad0, ad1, ad2, ad3, ad4, ad5)]).astype(jnp.bfloat16)
    b_all = jnp.stack([b0, b1, b2, b3, b4, b5]).reshape(_NLAYERS, 1, _NHID)

    out = pl.pallas_call(
        _gnn_graph_kernel,
        grid=(_B // _G,),
        in_specs=[
            pl.BlockSpec((_G, 1, _EPG), lambda g: (g, 0, 0)),
            pl.BlockSpec((_G, 1, _EPG), lambda g: (g, 0, 0)),
            pl.BlockSpec((_G, _NPG, _NINP), lambda g: (g, 0, 0)),
            pl.BlockSpec((_NLAYERS, _NINP, _NHID), lambda g: (0, 0, 0)),
            pl.BlockSpec((_NLAYERS, _NHID, _NHEADS), lambda g: (0, 0, 0)),
            pl.BlockSpec((_NLAYERS, _NHID, _NHEADS), lambda g: (0, 0, 0)),
            pl.BlockSpec((_NLAYERS, 1, _NHID), lambda g: (0, 0, 0)),
        ],
        out_specs=pl.BlockSpec((_G, 1, _NHID), lambda g: (g, 0, 0)),
        out_shape=jax.ShapeDtypeStruct((_B, 1, _NHID), jnp.float32),
        scratch_shapes=[pltpu.VMEM((_G, _NPG, _NHID), jnp.float32)],
        compiler_params=pltpu.CompilerParams(
            dimension_semantics=('parallel',)),
    )(src, dst, x, w_all, as_all, ad_all, b_all)
    return out.reshape(_B, _NHID)


# speedup vs baseline: 1.9367x; 1.9367x over previous
import jax
import jax.numpy as jnp
from jax.experimental import pallas as pl
from jax.experimental.pallas import tpu as pltpu

_NTOKEN = 50000
_NINP = 256
_NHID = 256
_NHEADS = 8
_NLAYERS = 6
_B = 100
_NPG = 100
_EPG = 1600
_HD = _NHID // _NHEADS
_G = 5


def _gnn_graph_kernel(src_ref, dst_ref, x_ref, w_ref, as_ref, ad_ref, b_ref,
                      out_ref, acc_ref):

    src = src_ref[...]
    dst = dst_ref[...]

    node_iota = jax.lax.broadcasted_iota(jnp.int32, (_G, _NPG, _EPG), 1)
    st = (node_iota == src).astype(jnp.bfloat16)
    dt = (node_iota == dst).astype(jnp.bfloat16)
    cnt = jax.lax.dot_general(dt, st, (((2,), (2,)), ((0,), (0,))),
                              preferred_element_type=jnp.float32)
    eye_r = jax.lax.broadcasted_iota(jnp.int32, (_G, _NPG, _NPG), 1)
    eye_c = jax.lax.broadcasted_iota(jnp.int32, (_G, _NPG, _NPG), 2)
    cnt = cnt + (eye_r == eye_c).astype(jnp.float32)

    ones_col = jnp.ones((_NPG, 1), dtype=jnp.bfloat16)
    ones_row = jnp.ones((1, _NPG), dtype=jnp.bfloat16)
    xs = [x_ref[gi] for gi in range(_G)]
    for l in range(_NLAYERS):
        w = w_ref[l]
        a_s = as_ref[l]
        a_d = ad_ref[l]
        for gi in range(_G):
            x = xs[gi]
            if l > 0:
                x = jnp.maximum(x, 0.0)
            h = jnp.dot(x.astype(jnp.bfloat16), w,
                        preferred_element_type=jnp.float32)
            hb = h.astype(jnp.bfloat16)
            als_t = jax.lax.dot_general(a_s, hb, (((0,), (1,)), ((), ())),
                                        preferred_element_type=jnp.float32)
            ald = jnp.dot(hb, a_d, preferred_element_type=jnp.float32)
            for hd_i in range(_NHEADS):
                e = ald[:, hd_i:hd_i + 1] + als_t[hd_i:hd_i + 1, :]
                e = jnp.maximum(e, 0.2 * e)
                ee = (cnt[gi] * jnp.exp(e)).astype(jnp.bfloat16)
                h_aug = jnp.concatenate(
                    [hb[:, hd_i * _HD:(hd_i + 1) * _HD], ones_col], axis=1)
                agg = jnp.dot(ee, h_aug, preferred_element_type=jnp.float32)
                acc_ref[gi, :, hd_i * _HD:(hd_i + 1) * _HD] = (
                    agg[:, :_HD] / agg[:, _HD:_HD + 1])
            xs[gi] = acc_ref[gi] + b_ref[l]

    for gi in range(_G):
        out_ref[gi] = jnp.sum(xs[gi], axis=0, keepdims=True) * (1.0 / _NPG)


def kernel(nodes, edges, emb,
           W0, as0, ad0, b0, W1, as1, ad1, b1, W2, as2, ad2, b2,
           W3, as3, ad3, b3, W4, as4, ad4, b4, W5, as5, ad5, b5):
    x = emb[nodes.reshape(-1)].reshape(_B, _NPG, _NINP)
    src = edges[:, 0::2].reshape(_B, 1, _EPG)
    dst = edges[:, 1::2].reshape(_B, 1, _EPG)

    w_all = jnp.stack([W0, W1, W2, W3, W4, W5]).astype(jnp.bfloat16)
    eye_h = jnp.eye(_NHEADS, dtype=jnp.float32)
    as_all = jnp.stack([
        jnp.einsum('hj,hk->hjk', a, eye_h).reshape(_NHID, _NHEADS)
        for a in (as0, as1, as2, as3, as4, as5)]).astype(jnp.bfloat16)
    ad_all = jnp.stack([
        jnp.einsum('hj,hk->hjk', a, eye_h).reshape(_NHID, _NHEADS)
        for a in (ad0, ad1, ad2, ad3, ad4, ad5)]).astype(jnp.bfloat16)
    b_all = jnp.stack([b0, b1, b2, b3, b4, b5]).reshape(_NLAYERS, 1, _NHID)

    out = pl.pallas_call(
        _gnn_graph_kernel,
        grid=(_B // _G,),
        in_specs=[
            pl.BlockSpec((_G, 1, _EPG), lambda g: (g, 0, 0)),
            pl.BlockSpec((_G, 1, _EPG), lambda g: (g, 0, 0)),
            pl.BlockSpec((_G, _NPG, _NINP), lambda g: (g, 0, 0)),
            pl.BlockSpec((_NLAYERS, _NINP, _NHID), lambda g: (0, 0, 0)),
            pl.BlockSpec((_NLAYERS, _NHID, _NHEADS), lambda g: (0, 0, 0)),
            pl.BlockSpec((_NLAYERS, _NHID, _NHEADS), lambda g: (0, 0, 0)),
            pl.BlockSpec((_NLAYERS, 1, _NHID), lambda g: (0, 0, 0)),
        ],
        out_specs=pl.BlockSpec((_G, 1, _NHID), lambda g: (g, 0, 0)),
        out_shape=jax.ShapeDtypeStruct((_B, 1, _NHID), jnp.float32),
        scratch_shapes=[pltpu.VMEM((_G, _NPG, _NHID), jnp.float32)],
        compiler_params=pltpu.CompilerParams(
            dimension_semantics=('parallel',)),
    )(src, dst, x, w_all, as_all, ad_all, b_all)
    return out.reshape(_B, _NHID)
